# Initial kernel scaffold; baseline (speedup 1.0000x reference)
import jax
import jax.numpy as jnp
from jax.experimental import pallas as pl


def kernel(detections, gt_xywh, gt_class_labels, gt_nearest_idx, z, r):
    raise NotImplementedError("write your pallas kernel here")


# speedup vs baseline: 2.8029x; 2.8029x over previous
import functools

import jax
import jax.numpy as jnp
from jax import lax
from jax.experimental import pallas as pl
from jax.experimental.pallas import tpu as pltpu
from jax.experimental.pallas import tpu_sc as plsc

N = 5000
G = 100
C = 80
ROW = 5 + C
NC, NS, L = 2, 16, 16
NW = NC * NS
RPW = 160
BASE_LAST = N - RPW
NGRP = RPW // L


def _sc_body(det_hbm, xywh_hbm, lab_hbm, idx_hbm, maxv_hbm, dist_hbm,
             det_v, idx_v, lab_v, xywh_v, maxv_v, dist_v):
    wid = lax.axis_index("s") * NC + lax.axis_index("c")
    base = jnp.minimum(wid * RPW, BASE_LAST)

    pltpu.sync_copy(det_hbm.at[pl.ds(base * ROW, RPW * ROW)], det_v)
    pltpu.sync_copy(idx_hbm.at[pl.ds(base, RPW)], idx_v)
    pltpu.sync_copy(lab_hbm, lab_v)
    pltpu.sync_copy(xywh_hbm, xywh_v)

    zeros = jnp.zeros((L,), jnp.float32)
    lane = lax.iota(jnp.int32, L)

    for g in range(NGRP):
        g0 = g * L
        idx16 = idx_v[pl.ds(g0, L)]
        lab16 = plsc.load_gather(lab_v, [idx16])
        rowbase = (lane + g0) * ROW

        plsc.store_scatter(det_v, [rowbase + (5 + lab16)], zeros)

        a = rowbase + 5
        acc = zeros
        for _ in range(C):
            v = plsc.load_gather(det_v, [a])
            acc = jnp.maximum(acc, v)
            a = a + 1
        maxv_v[pl.ds(g0, L)] = acc

        xb = idx16 * 4
        d = zeros
        for c in range(4):
            av = plsc.load_gather(det_v, [rowbase + c])
            bv = plsc.load_gather(xywh_v, [xb + c])
            t = av - bv
            d = d + t * t
        dist_v[pl.ds(g0, L)] = d

    pltpu.sync_copy(maxv_v, maxv_hbm.at[pl.ds(base, RPW)])
    pltpu.sync_copy(dist_v, dist_hbm.at[pl.ds(base, RPW)])


_sc_call = functools.partial(
    pl.kernel,
    mesh=plsc.VectorSubcoreMesh(core_axis_name="c", subcore_axis_name="s"),
    out_type=[
        jax.ShapeDtypeStruct((N,), jnp.float32),
        jax.ShapeDtypeStruct((N,), jnp.float32),
    ],
    scratch_types=[
        pltpu.VMEM((RPW * ROW,), jnp.float32),
        pltpu.VMEM((RPW,), jnp.int32),
        pltpu.VMEM((G,), jnp.int32),
        pltpu.VMEM((G * 4,), jnp.float32),
        pltpu.VMEM((RPW,), jnp.float32),
        pltpu.VMEM((RPW,), jnp.float32),
    ],
    compiler_params=pltpu.CompilerParams(needs_layout_passes=False),
)(_sc_body)


def _tc_body(maxv_ref, dist_ref, z_ref, r_ref, out_ref):
    lm = jnp.log(maxv_ref[...])
    s_cls = jnp.sum((z_ref[...] + r_ref[...]) * lm)
    s_box = jnp.sum(z_ref[...] * dist_ref[...])
    out_ref[0, 0] = jnp.exp(-s_box) - s_cls


_tc_call = pl.pallas_call(
    _tc_body,
    out_shape=jax.ShapeDtypeStruct((1, 1), jnp.float32),
    out_specs=pl.BlockSpec(memory_space=pltpu.SMEM),
)


@jax.jit
def kernel(detections, gt_xywh, gt_class_labels, gt_nearest_idx, z, r):
    maxv, dist = _sc_call(
        detections.reshape(-1),
        gt_xywh.reshape(-1),
        gt_class_labels,
        gt_nearest_idx,
    )

    pad = NW * RPW - N
    ones_pad = jnp.ones((pad,), jnp.float32)
    zeros_pad = jnp.zeros((pad,), jnp.float32)
    maxv2 = jnp.concatenate([maxv, ones_pad]).reshape(40, 128)
    dist2 = jnp.concatenate([dist, zeros_pad]).reshape(40, 128)
    z2 = jnp.concatenate([z, zeros_pad]).reshape(40, 128)
    r2 = jnp.concatenate([r, zeros_pad]).reshape(40, 128)

    loss = _tc_call(maxv2, dist2, z2, r2)
    return loss.reshape(1)


# speedup vs baseline: 2.9246x; 1.0434x over previous
import functools

import jax
import jax.numpy as jnp
from jax import lax
from jax.experimental import pallas as pl
from jax.experimental.pallas import tpu as pltpu
from jax.experimental.pallas import tpu_sc as plsc

N = 5000
G = 100
C = 80
ROW = 5 + C
NC, NS, L = 2, 16, 16
NW = NC * NS
RPW = 160
BASE_LAST = N - RPW
NGRP = RPW // L


def _sc_body(det_hbm, xywh_hbm, lab_hbm, idx_hbm, maxv_hbm, dist_hbm,
             det_v, idx_v, lab_v, xywh_v, maxv_v, dist_v, sem):
    wid = lax.axis_index("s") * NC + lax.axis_index("c")
    base = jnp.minimum(wid * RPW, BASE_LAST)

    copies = [
        pltpu.async_copy(det_hbm.at[pl.ds(base * ROW, RPW * ROW)], det_v, sem),
        pltpu.async_copy(idx_hbm.at[pl.ds(base, RPW)], idx_v, sem),
        pltpu.async_copy(lab_hbm, lab_v, sem),
        pltpu.async_copy(xywh_hbm, xywh_v, sem),
    ]
    for cp in copies:
        cp.wait()

    zeros = jnp.zeros((L,), jnp.float32)
    lane = lax.iota(jnp.int32, L)

    for g in range(NGRP):
        g0 = g * L
        idx16 = idx_v[pl.ds(g0, L)]
        lab16 = plsc.load_gather(lab_v, [idx16])
        rowbase = (lane + g0) * ROW

        plsc.store_scatter(det_v, [rowbase + (5 + lab16)], zeros)

        a = rowbase + 5
        acc = zeros
        for _ in range(C):
            v = plsc.load_gather(det_v, [a])
            acc = jnp.maximum(acc, v)
            a = a + 1
        maxv_v[pl.ds(g0, L)] = acc

        xb = idx16 * 4
        d = zeros
        for c in range(4):
            av = plsc.load_gather(det_v, [rowbase + c])
            bv = plsc.load_gather(xywh_v, [xb + c])
            t = av - bv
            d = d + t * t
        dist_v[pl.ds(g0, L)] = d

    out_copies = [
        pltpu.async_copy(maxv_v, maxv_hbm.at[pl.ds(base, RPW)], sem),
        pltpu.async_copy(dist_v, dist_hbm.at[pl.ds(base, RPW)], sem),
    ]
    for cp in out_copies:
        cp.wait()


_sc_call = functools.partial(
    pl.kernel,
    mesh=plsc.VectorSubcoreMesh(core_axis_name="c", subcore_axis_name="s"),
    out_type=[
        jax.ShapeDtypeStruct((N,), jnp.float32),
        jax.ShapeDtypeStruct((N,), jnp.float32),
    ],
    scratch_types=[
        pltpu.VMEM((RPW * ROW,), jnp.float32),
        pltpu.VMEM((RPW,), jnp.int32),
        pltpu.VMEM((G,), jnp.int32),
        pltpu.VMEM((G * 4,), jnp.float32),
        pltpu.VMEM((RPW,), jnp.float32),
        pltpu.VMEM((RPW,), jnp.float32),
        pltpu.SemaphoreType.DMA,
    ],
    compiler_params=pltpu.CompilerParams(needs_layout_passes=False),
)(_sc_body)


def _tc_body(maxv_ref, dist_ref, z_ref, r_ref, out_ref):
    lm = jnp.log(maxv_ref[...])
    s_cls = jnp.sum((z_ref[...] + r_ref[...]) * lm)
    s_box = jnp.sum(z_ref[...] * dist_ref[...])
    out_ref[0, 0] = jnp.exp(-s_box) - s_cls


_tc_call = pl.pallas_call(
    _tc_body,
    out_shape=jax.ShapeDtypeStruct((1, 1), jnp.float32),
    out_specs=pl.BlockSpec(memory_space=pltpu.SMEM),
)


@jax.jit
def kernel(detections, gt_xywh, gt_class_labels, gt_nearest_idx, z, r):
    maxv, dist = _sc_call(
        detections.reshape(-1),
        gt_xywh.reshape(-1),
        gt_class_labels,
        gt_nearest_idx,
    )

    pad = NW * RPW - N
    ones_pad = jnp.ones((pad,), jnp.float32)
    zeros_pad = jnp.zeros((pad,), jnp.float32)
    maxv2 = jnp.concatenate([maxv, ones_pad]).reshape(40, 128)
    dist2 = jnp.concatenate([dist, zeros_pad]).reshape(40, 128)
    z2 = jnp.concatenate([z, zeros_pad]).reshape(40, 128)
    r2 = jnp.concatenate([r, zeros_pad]).reshape(40, 128)

    loss = _tc_call(maxv2, dist2, z2, r2)
    return loss.reshape(1)


# speedup vs baseline: 3.0918x; 1.0572x over previous
import functools

import jax
import jax.numpy as jnp
from jax import lax
from jax.experimental import pallas as pl
from jax.experimental.pallas import tpu as pltpu
from jax.experimental.pallas import tpu_sc as plsc

N = 5000
G = 100
C = 80
ROW = 5 + C
NC, NS, L = 2, 16, 16
NW = NC * NS
RPW = 160
BASE_LAST = N - RPW
NGRP = RPW // L


def _sc_body(det_hbm, xywh_hbm, lab_hbm, idx_hbm, maxv_hbm, dist_hbm,
             det_v, idx_v, lab_v, xywh_v, maxv_v, dist_v, sem):
    wid = lax.axis_index("s") * NC + lax.axis_index("c")
    base = jnp.minimum(wid * RPW, BASE_LAST)

    copies = [
        pltpu.async_copy(det_hbm.at[pl.ds(base * ROW, RPW * ROW)], det_v, sem),
        pltpu.async_copy(idx_hbm.at[pl.ds(base, RPW)], idx_v, sem),
        pltpu.async_copy(lab_hbm, lab_v, sem),
        pltpu.async_copy(xywh_hbm, xywh_v, sem),
    ]
    for cp in copies:
        cp.wait()

    zeros = jnp.zeros((L,), jnp.float32)
    lane = lax.iota(jnp.int32, L)

    for g in range(NGRP):
        g0 = g * L
        idx16 = idx_v[pl.ds(g0, L)]
        lab16 = plsc.load_gather(lab_v, [idx16])
        rowbase = (lane + g0) * ROW

        plsc.store_scatter(det_v, [rowbase + (5 + lab16)], zeros)

        a = rowbase + 5
        acc = zeros
        for _ in range(C):
            v = plsc.load_gather(det_v, [a])
            acc = jnp.maximum(acc, v)
            a = a + 1
        maxv_v[pl.ds(g0, L)] = acc

        xb = idx16 * 4
        d = zeros
        for c in range(4):
            av = plsc.load_gather(det_v, [rowbase + c])
            bv = plsc.load_gather(xywh_v, [xb + c])
            t = av - bv
            d = d + t * t
        dist_v[pl.ds(g0, L)] = d

    out_copies = [
        pltpu.async_copy(maxv_v, maxv_hbm.at[pl.ds(base, RPW)], sem),
        pltpu.async_copy(dist_v, dist_hbm.at[pl.ds(base, RPW)], sem),
    ]
    for cp in out_copies:
        cp.wait()


_sc_call = functools.partial(
    pl.kernel,
    mesh=plsc.VectorSubcoreMesh(core_axis_name="c", subcore_axis_name="s"),
    out_type=[
        jax.ShapeDtypeStruct((N,), jnp.float32),
        jax.ShapeDtypeStruct((N,), jnp.float32),
    ],
    scratch_types=[
        pltpu.VMEM((RPW * ROW,), jnp.float32),
        pltpu.VMEM((RPW,), jnp.int32),
        pltpu.VMEM((G,), jnp.int32),
        pltpu.VMEM((G * 4,), jnp.float32),
        pltpu.VMEM((RPW,), jnp.float32),
        pltpu.VMEM((RPW,), jnp.float32),
        pltpu.SemaphoreType.DMA,
    ],
    compiler_params=pltpu.CompilerParams(needs_layout_passes=False),
)(_sc_body)


def _tc_body(maxv_ref, dist_ref, z_ref, r_ref, out_ref):
    lm = jnp.log(maxv_ref[...])
    s_cls = jnp.sum((z_ref[...] + r_ref[...]) * lm)
    s_box = jnp.sum(z_ref[...] * dist_ref[...])
    out_ref[0, 0] = jnp.exp(-s_box) - s_cls


_tc_call = pl.pallas_call(
    _tc_body,
    out_shape=jax.ShapeDtypeStruct((1, 1), jnp.float32),
    out_specs=pl.BlockSpec(memory_space=pltpu.SMEM),
)


@jax.jit
def kernel(detections, gt_xywh, gt_class_labels, gt_nearest_idx, z, r):
    maxv, dist = _sc_call(
        detections.reshape(-1),
        gt_xywh.reshape(-1),
        gt_class_labels,
        gt_nearest_idx,
    )

    loss = _tc_call(maxv, dist, z, r)
    return loss.reshape(1)
